# baseline (speedup 1.0000x reference)
import functools

import jax
import jax.numpy as jnp
from jax import lax
from jax.experimental import pallas as pl
from jax.experimental.pallas import tpu as pltpu
from jax.experimental.pallas import tpu_sc as plsc

NC = 2
NS = 16
NW = NC * NS

BLK = 128
N_ATOMS = 100000
NB_FULL = N_ATOMS // BLK
TAIL = N_ATOMS - NB_FULL * BLK
TAIL_OFF = NB_FULL * BLK
BPW = -(-NB_FULL // NW)
STAGE = BPW * BLK
NBUF = 3


def kernel(z, tag, base_table, tag_table):
    n, d_base = N_ATOMS, base_table.shape[1]
    d_tag = tag_table.shape[1]
    d = d_base + d_tag
    zi = z.astype(jnp.int32)
    ti = tag.astype(jnp.int32)

    mesh = plsc.VectorSubcoreMesh(
        core_axis_name="c", subcore_axis_name="s",
        num_cores=NC, num_subcores=NS)

    @functools.partial(
        pl.kernel,
        out_type=jax.ShapeDtypeStruct((n, d), jnp.float32),
        mesh=mesh,
        compiler_params=pltpu.CompilerParams(use_tc_tiling_on_sc=False),
        scratch_types=[
            pltpu.VMEM((STAGE,), jnp.int32),
            pltpu.VMEM((STAGE,), jnp.int32),
            [pltpu.VMEM((BLK, d_base), jnp.float32) for _ in range(NBUF)],
            [pltpu.VMEM((BLK, d_tag), jnp.float32) for _ in range(NBUF)],
            [pltpu.SemaphoreType.DMA for _ in range(NBUF)],
            [pltpu.SemaphoreType.DMA for _ in range(NBUF)],
            pltpu.VMEM((TAIL,), jnp.int32),
            pltpu.VMEM((TAIL,), jnp.int32),
            pltpu.VMEM((TAIL, d_base), jnp.float32),
            pltpu.VMEM((TAIL, d_tag), jnp.float32),
            pltpu.SemaphoreType.DMA,
        ],
    )
    def sc_kernel(z_hbm, t_hbm, base_hbm, tagtab_hbm,
                  out_hbm, zv, tv, bufs_a, bufs_b, gsems, wsems,
                  ztv, ttv, tbuf_a, tbuf_b, tsem):
        wid = lax.axis_index("s") * NC + lax.axis_index("c")
        lo = (wid * NB_FULL) >> 5
        base_atom = pl.multiple_of(lo * BLK, BLK)
        pltpu.sync_copy(z_hbm.at[pl.ds(base_atom, STAGE)], zv)
        pltpu.sync_copy(t_hbm.at[pl.ds(base_atom, STAGE)], tv)

        def start_gather(t):
            b = t % NBUF
            ca = pltpu.async_copy(
                base_hbm.at[zv.at[pl.ds(t * BLK, BLK)]], bufs_a[b], gsems[b])
            cb = pltpu.async_copy(
                tagtab_hbm.at[tv.at[pl.ds(t * BLK, BLK)]], bufs_b[b], gsems[b])
            return ca, cb

        def start_write(t):
            b = t % NBUF
            row0 = pl.multiple_of(base_atom + t * BLK, BLK)
            wa = pltpu.async_copy(
                bufs_a[b], out_hbm.at[pl.ds(row0, BLK), pl.ds(0, d_base)],
                wsems[b])
            wb = pltpu.async_copy(
                bufs_b[b], out_hbm.at[pl.ds(row0, BLK), pl.ds(d_base, d_tag)],
                wsems[b])
            return wa, wb

        gathers = {}
        writes = {}
        gathers[0] = start_gather(0)
        gathers[1] = start_gather(1)
        for t in range(BPW):
            ga, gb = gathers.pop(t)
            ga.wait()
            gb.wait()
            if t >= 1:
                wa, wb = writes.pop(t - 1)
                wa.wait()
                wb.wait()
            if t + 2 < BPW:
                gathers[t + 2] = start_gather(t + 2)
            writes[t] = start_write(t)
        wa, wb = writes.pop(BPW - 1)
        wa.wait()
        wb.wait()

        @pl.when(wid == 0)
        def _tail():
            pltpu.sync_copy(z_hbm.at[pl.ds(TAIL_OFF, TAIL)], ztv)
            pltpu.sync_copy(t_hbm.at[pl.ds(TAIL_OFF, TAIL)], ttv)
            cp_a = pltpu.async_copy(base_hbm.at[ztv], tbuf_a, tsem)
            cp_b = pltpu.async_copy(tagtab_hbm.at[ttv], tbuf_b, tsem)
            cp_a.wait()
            cp_b.wait()
            pltpu.sync_copy(
                tbuf_a, out_hbm.at[pl.ds(TAIL_OFF, TAIL), pl.ds(0, d_base)])
            pltpu.sync_copy(
                tbuf_b, out_hbm.at[pl.ds(TAIL_OFF, TAIL), pl.ds(d_base, d_tag)])

    return sc_kernel(zi, ti, base_table, tag_table)


# speedup vs baseline: 4.7144x; 4.7144x over previous
import functools

import jax
import jax.numpy as jnp
from jax import lax
from jax.experimental import pallas as pl
from jax.experimental.pallas import tpu as pltpu
from jax.experimental.pallas import tpu_sc as plsc

NC = 2
NS = 16
NW = NC * NS
L = 16

BLK = 128
N_ATOMS = 100000
NB_FULL = N_ATOMS // BLK
TAIL = N_ATOMS - NB_FULL * BLK
TAIL_OFF = NB_FULL * BLK
BPW = -(-NB_FULL // NW)
STAGE = BPW * BLK
NBUF = 3

T_ROWS = 384
ROWS_PER_TILE = T_ROWS // NS
ZPT = ROWS_PER_TILE // 3
BUILD_TILES = -(-303 // ROWS_PER_TILE)


def kernel(z, tag, base_table, tag_table):
    n, d_base = N_ATOMS, base_table.shape[1]
    d_tag = tag_table.shape[1]
    d = d_base + d_tag
    zi = z.astype(jnp.int32)
    ti = tag.astype(jnp.int32)
    base_pad = jnp.pad(base_table, ((0, BUILD_TILES * ZPT - base_table.shape[0]), (0, 0)))

    mesh = plsc.VectorSubcoreMesh(
        core_axis_name="c", subcore_axis_name="s",
        num_cores=NC, num_subcores=NS)

    @functools.partial(
        pl.kernel,
        out_type=(jax.ShapeDtypeStruct((n, d), jnp.float32),
                  jax.ShapeDtypeStruct((T_ROWS, d), jnp.float32)),
        mesh=mesh,
        compiler_params=pltpu.CompilerParams(use_tc_tiling_on_sc=False),
        scratch_types=[
            pltpu.VMEM((STAGE,), jnp.int32),
            pltpu.VMEM((STAGE,), jnp.int32),
            pltpu.VMEM((STAGE,), jnp.int32),
            pltpu.VMEM((ZPT, 224), jnp.float32),
            pltpu.VMEM((3, 32), jnp.float32),
            pltpu.VMEM((ROWS_PER_TILE, 256), jnp.float32),
            [pltpu.VMEM((BLK, d), jnp.float32) for _ in range(NBUF)],
            [pltpu.SemaphoreType.DMA for _ in range(NBUF)],
            [pltpu.SemaphoreType.DMA for _ in range(NBUF)],
            pltpu.VMEM((TAIL,), jnp.int32),
            pltpu.VMEM((TAIL, d), jnp.float32),
            pltpu.SemaphoreType.DMA,
        ],
    )
    def sc_kernel(z_hbm, t_hbm, base_hbm, tagtab_hbm,
                  out_hbm, ctable_hbm, zv, tv, civ, basev, tagv, bld,
                  bufs, gsems, wsems, tiv, tbuf, tsem):
        cid = lax.axis_index("c")
        sid = lax.axis_index("s")
        wid = sid * NC + cid
        lo = (wid * NB_FULL) >> 5
        base_atom = pl.multiple_of(lo * BLK, BLK)

        pltpu.sync_copy(z_hbm.at[pl.ds(base_atom, STAGE)], zv)
        pltpu.sync_copy(t_hbm.at[pl.ds(base_atom, STAGE)], tv)
        for k in range(STAGE // L):
            s = pl.ds(k * L, L)
            civ[s] = zv[s] * 3 + tv[s]

        @pl.when(sid < BUILD_TILES)
        def _build():
            zrow0 = pl.multiple_of(sid * ZPT, ZPT)
            pltpu.sync_copy(base_hbm.at[pl.ds(zrow0, ZPT), :], basev)
            pltpu.sync_copy(tagtab_hbm, tagv)
            for r in range(ZPT):
                for rep in range(3):
                    row = 3 * r + rep
                    for c in range(d_base // L):
                        bld[row, pl.ds(c * L, L)] = basev[r, pl.ds(c * L, L)]
                    for c in range(d_tag // L):
                        bld[row, pl.ds(d_base + c * L, L)] = (
                            tagv[rep, pl.ds(c * L, L)])
            crow0 = pl.multiple_of(sid * ROWS_PER_TILE, ROWS_PER_TILE)
            pltpu.sync_copy(bld, ctable_hbm.at[pl.ds(crow0, ROWS_PER_TILE), :])

        plsc.subcore_barrier()

        def start_gather(t):
            b = t % NBUF
            return pltpu.async_copy(
                ctable_hbm.at[civ.at[pl.ds(t * BLK, BLK)]], bufs[b], gsems[b])

        def start_write(t):
            b = t % NBUF
            row0 = pl.multiple_of(base_atom + t * BLK, BLK)
            return pltpu.async_copy(
                bufs[b], out_hbm.at[pl.ds(row0, BLK), :], wsems[b])

        gathers = {0: start_gather(0), 1: start_gather(1)}
        writes = {}
        for t in range(BPW):
            gathers.pop(t).wait()
            if t >= 1:
                writes.pop(t - 1).wait()
            if t + 2 < BPW:
                gathers[t + 2] = start_gather(t + 2)
            writes[t] = start_write(t)
        writes.pop(BPW - 1).wait()

        @pl.when(wid == 0)
        def _tail():
            pltpu.sync_copy(z_hbm.at[pl.ds(TAIL_OFF, TAIL)], zv.at[pl.ds(0, TAIL)])
            pltpu.sync_copy(t_hbm.at[pl.ds(TAIL_OFF, TAIL)], tv.at[pl.ds(0, TAIL)])
            for k in range(TAIL // L):
                s = pl.ds(k * L, L)
                tiv[s] = zv[s] * 3 + tv[s]
            pltpu.async_copy(ctable_hbm.at[tiv], tbuf, tsem).wait()
            pltpu.sync_copy(tbuf, out_hbm.at[pl.ds(TAIL_OFF, TAIL), :])

    out, _ = sc_kernel(zi, ti, base_pad, tag_table)
    return out
